# baseline (speedup 1.0000x reference)
import jax
import jax.numpy as jnp
from jax.experimental import pallas as pl
from jax.experimental.pallas import tpu as pltpu

N = 16384
C = 10
GRID = 4
BLK = N // GRID


def _onehot_block(lbl_ref, out_ref):
    lbl = lbl_ref[...]
    classes = jax.lax.broadcasted_iota(jnp.int32, (C, BLK), 0)
    out_ref[...] = jnp.where(lbl[None, :] == classes, 1.0, 0.0).astype(
        jnp.float32
    )


_onehot_tc = pl.pallas_call(
    _onehot_block,
    grid=(GRID,),
    in_specs=[pl.BlockSpec((BLK,), lambda i: (i,))],
    out_specs=pl.BlockSpec((C, BLK), lambda i: (0, i)),
    out_shape=jax.ShapeDtypeStruct((C, N), jnp.float32),
    compiler_params=pltpu.CompilerParams(
        dimension_semantics=("parallel",),
    ),
)


@jax.jit
def kernel(label):
    return _onehot_tc(label).T


# speedup vs baseline: 1.5510x; 1.5510x over previous
import jax
import jax.numpy as jnp
from jax.experimental import pallas as pl
from jax.experimental.pallas import tpu as pltpu

N = 16384
C = 10
BLK = N


def _onehot_block(lbl_ref, out_ref):
    lbl = lbl_ref[...]
    classes = jax.lax.broadcasted_iota(jnp.int32, (C, BLK), 0)
    out_ref[...] = jnp.where(lbl[None, :] == classes, 1.0, 0.0).astype(
        jnp.float32
    )


_onehot_tc = pl.pallas_call(
    _onehot_block,
    out_shape=jax.ShapeDtypeStruct((C, N), jnp.float32),
)


@jax.jit
def kernel(label):
    return _onehot_tc(label).T


# speedup vs baseline: 1.6117x; 1.0391x over previous
import jax
import jax.numpy as jnp
from jax.experimental import pallas as pl
from jax.experimental.pallas import tpu as pltpu

N = 16384
C = 10
CHUNKS = 4
CW = N // CHUNKS


def _onehot_block(lbl_ref, out_hbm, stage, sem):
    classes = jax.lax.broadcasted_iota(jnp.int32, (C, CW), 0)
    for k in range(CHUNKS):
        lbl = lbl_ref[pl.ds(k * CW, CW)]
        stage[:, pl.ds(k * CW, CW)] = jnp.where(
            lbl[None, :] == classes, 1.0, 0.0
        ).astype(jnp.float32)
        pltpu.make_async_copy(
            stage.at[:, pl.ds(k * CW, CW)],
            out_hbm.at[:, pl.ds(k * CW, CW)],
            sem,
        ).start()
    for k in range(CHUNKS):
        pltpu.make_async_copy(
            stage.at[:, pl.ds(k * CW, CW)],
            out_hbm.at[:, pl.ds(k * CW, CW)],
            sem,
        ).wait()


_onehot_tc = pl.pallas_call(
    _onehot_block,
    out_specs=pl.BlockSpec(memory_space=pltpu.MemorySpace.HBM),
    out_shape=jax.ShapeDtypeStruct((C, N), jnp.float32),
    scratch_shapes=[
        pltpu.VMEM((C, N), jnp.float32),
        pltpu.SemaphoreType.DMA,
    ],
)


@jax.jit
def kernel(label):
    return _onehot_tc(label).T
